# Initial kernel scaffold; baseline (speedup 1.0000x reference)
import jax
import jax.numpy as jnp
from jax.experimental import pallas as pl


def kernel(animal_id, item_id, hp, atk, emb_animal, emb_item, W_lin, b_lin):
    raise NotImplementedError("write your pallas kernel here")


# speedup vs baseline: 1.9279x; 1.9279x over previous
import functools

import jax
import jax.numpy as jnp
from jax import lax
from jax.experimental import pallas as pl
from jax.experimental.pallas import tpu as pltpu
from jax.experimental.pallas import tpu_sc as plsc

B = 16384
NC, NS, L = 2, 16, 16
NW = NC * NS
BPW = B // NW
GROUPS = BPW // L

_mesh = plsc.VectorSubcoreMesh(core_axis_name="c", subcore_axis_name="s")


@functools.partial(
    pl.kernel,
    out_type=(
        jax.ShapeDtypeStruct((B * 5,), jnp.float32),
        jax.ShapeDtypeStruct((B * 3,), jnp.float32),
        jax.ShapeDtypeStruct((B * 2,), jnp.float32),
    ),
    mesh=_mesh,
    scratch_types=(
        pltpu.VMEM((BPW,), jnp.int32),
        pltpu.VMEM((BPW,), jnp.int32),
        pltpu.VMEM((BPW,), jnp.float32),
        pltpu.VMEM((BPW,), jnp.float32),
        pltpu.VMEM((400,), jnp.float32),
        pltpu.VMEM((64,), jnp.float32),
        pltpu.VMEM((96,), jnp.float32),
        pltpu.VMEM((BPW * 5,), jnp.float32),
        pltpu.VMEM((BPW * 3,), jnp.float32),
        pltpu.VMEM((BPW * 2,), jnp.float32),
    ),
    compiler_params=pltpu.CompilerParams(needs_layout_passes=False),
)
def _sc_embed(aid_h, iid_h, hp_h, atk_h, taba_h, tabi_h, wb_h,
              outa_h, outi_h, outs_h,
              aid_v, iid_v, hp_v, atk_v, taba_v, tabi_v, wb_v,
              outa_v, outi_v, outs_v):
    wid = lax.axis_index("s") * NC + lax.axis_index("c")
    base = wid * BPW

    pltpu.sync_copy(aid_h.at[pl.ds(base, BPW)], aid_v)
    pltpu.sync_copy(iid_h.at[pl.ds(base, BPW)], iid_v)
    pltpu.sync_copy(hp_h.at[pl.ds(base, BPW)], hp_v)
    pltpu.sync_copy(atk_h.at[pl.ds(base, BPW)], atk_v)
    pltpu.sync_copy(taba_h, taba_v)
    pltpu.sync_copy(tabi_h, tabi_v)
    pltpu.sync_copy(wb_h, wb_v)

    w00 = wb_v[pl.ds(0, L)]
    w01 = wb_v[pl.ds(L, L)]
    w10 = wb_v[pl.ds(2 * L, L)]
    w11 = wb_v[pl.ds(3 * L, L)]
    b0 = wb_v[pl.ds(4 * L, L)]
    b1 = wb_v[pl.ds(5 * L, L)]
    iota = lax.iota(jnp.int32, L)

    for g in range(GROUPS):
        off = g * L
        pos = iota + off
        aidx = aid_v[pl.ds(off, L)] * 5
        pa = pos * 5
        for j in range(5):
            plsc.store_scatter(outa_v, [pa + j],
                               plsc.load_gather(taba_v, [aidx + j]))
        iidx = iid_v[pl.ds(off, L)] * 3
        pi = pos * 3
        for j in range(3):
            plsc.store_scatter(outi_v, [pi + j],
                               plsc.load_gather(tabi_v, [iidx + j]))
        h = hp_v[pl.ds(off, L)]
        a = atk_v[pl.ds(off, L)]
        ps = pos * 2
        plsc.store_scatter(outs_v, [ps], h * w00 + a * w01 + b0)
        plsc.store_scatter(outs_v, [ps + 1], h * w10 + a * w11 + b1)

    pltpu.sync_copy(outa_v, outa_h.at[pl.ds(base * 5, BPW * 5)])
    pltpu.sync_copy(outi_v, outi_h.at[pl.ds(base * 3, BPW * 3)])
    pltpu.sync_copy(outs_v, outs_h.at[pl.ds(base * 2, BPW * 2)])


def kernel(animal_id, item_id, hp, atk, emb_animal, emb_item, W_lin, b_lin):
    taba = emb_animal.reshape(-1)
    tabi = jnp.pad(emb_item.reshape(-1), (0, 4))
    wb = jnp.broadcast_to(
        jnp.concatenate([W_lin.reshape(-1), b_lin])[:, None], (6, L)
    ).reshape(-1)
    outa, outi, outs = _sc_embed(
        animal_id.astype(jnp.int32), item_id.astype(jnp.int32),
        hp, atk, taba, tabi, wb)
    return (outa.reshape(B, 5), outi.reshape(B, 3), outs.reshape(B, 2))


# speedup vs baseline: 1.9488x; 1.0108x over previous
import functools

import jax
import jax.numpy as jnp
from jax import lax
from jax.experimental import pallas as pl
from jax.experimental.pallas import tpu as pltpu
from jax.experimental.pallas import tpu_sc as plsc

B = 16384
NC, NS, L = 2, 16, 16
NW = NC * NS
BPW = B // NW
GROUPS = BPW // L

_mesh = plsc.VectorSubcoreMesh(core_axis_name="c", subcore_axis_name="s")


@functools.partial(
    pl.kernel,
    out_type=(
        jax.ShapeDtypeStruct((B * 5,), jnp.float32),
        jax.ShapeDtypeStruct((B * 3,), jnp.float32),
        jax.ShapeDtypeStruct((B * 2,), jnp.float32),
    ),
    mesh=_mesh,
    scratch_types=(
        pltpu.VMEM((BPW,), jnp.int32),
        pltpu.VMEM((BPW,), jnp.int32),
        pltpu.VMEM((BPW,), jnp.float32),
        pltpu.VMEM((BPW,), jnp.float32),
        pltpu.VMEM((80, 5), jnp.float32),
        pltpu.VMEM((20, 3), jnp.float32),
        pltpu.VMEM((2, 2), jnp.float32),
        pltpu.VMEM((2,), jnp.float32),
        pltpu.VMEM((BPW * 5,), jnp.float32),
        pltpu.VMEM((BPW * 3,), jnp.float32),
        pltpu.VMEM((BPW * 2,), jnp.float32),
        pltpu.SemaphoreType.DMA,
    ),
    compiler_params=pltpu.CompilerParams(needs_layout_passes=False),
)
def _sc_embed(aid_h, iid_h, hp_h, atk_h, taba_h, tabi_h, w_h, b_h,
              outa_h, outi_h, outs_h,
              aid_v, iid_v, hp_v, atk_v, taba_v, tabi_v, w_v, b_v,
              outa_v, outi_v, outs_v, sem):
    wid = lax.axis_index("s") * NC + lax.axis_index("c")
    base = wid * BPW

    copies = [
        pltpu.async_copy(aid_h.at[pl.ds(base, BPW)], aid_v, sem),
        pltpu.async_copy(iid_h.at[pl.ds(base, BPW)], iid_v, sem),
        pltpu.async_copy(hp_h.at[pl.ds(base, BPW)], hp_v, sem),
        pltpu.async_copy(atk_h.at[pl.ds(base, BPW)], atk_v, sem),
        pltpu.async_copy(taba_h, taba_v, sem),
        pltpu.async_copy(tabi_h, tabi_v, sem),
        pltpu.async_copy(w_h, w_v, sem),
        pltpu.async_copy(b_h, b_v, sem),
    ]
    for c in copies:
        c.wait()

    zero = jnp.zeros((L,), jnp.int32)
    one = zero + 1
    w00 = plsc.load_gather(w_v, [zero, zero])
    w01 = plsc.load_gather(w_v, [zero, one])
    w10 = plsc.load_gather(w_v, [one, zero])
    w11 = plsc.load_gather(w_v, [one, one])
    b0 = plsc.load_gather(b_v, [zero])
    b1 = plsc.load_gather(b_v, [one])
    iota = lax.iota(jnp.int32, L)

    for g in range(GROUPS):
        off = g * L
        pos = iota + off
        aidx = aid_v[pl.ds(off, L)]
        pa = pos * 5
        for j in range(5):
            plsc.store_scatter(outa_v, [pa + j],
                               plsc.load_gather(taba_v, [aidx, zero + j]))
        iidx = iid_v[pl.ds(off, L)]
        pi = pos * 3
        for j in range(3):
            plsc.store_scatter(outi_v, [pi + j],
                               plsc.load_gather(tabi_v, [iidx, zero + j]))
        h = hp_v[pl.ds(off, L)]
        a = atk_v[pl.ds(off, L)]
        ps = pos * 2
        plsc.store_scatter(outs_v, [ps], h * w00 + a * w01 + b0)
        plsc.store_scatter(outs_v, [ps + 1], h * w10 + a * w11 + b1)

    out_copies = [
        pltpu.async_copy(outa_v, outa_h.at[pl.ds(base * 5, BPW * 5)], sem),
        pltpu.async_copy(outi_v, outi_h.at[pl.ds(base * 3, BPW * 3)], sem),
        pltpu.async_copy(outs_v, outs_h.at[pl.ds(base * 2, BPW * 2)], sem),
    ]
    for c in out_copies:
        c.wait()


def kernel(animal_id, item_id, hp, atk, emb_animal, emb_item, W_lin, b_lin):
    outa, outi, outs = _sc_embed(animal_id, item_id, hp, atk,
                                 emb_animal, emb_item, W_lin, b_lin)
    return (outa.reshape(B, 5), outi.reshape(B, 3), outs.reshape(B, 2))


# speedup vs baseline: 1.9933x; 1.0229x over previous
import functools

import jax
import jax.numpy as jnp
from jax import lax
from jax.experimental import pallas as pl
from jax.experimental.pallas import tpu as pltpu
from jax.experimental.pallas import tpu_sc as plsc

B = 16384
NC, NS, L = 2, 16, 16
NW = NC * NS
BPW = B // NW
GROUPS = BPW // L

_mesh = plsc.VectorSubcoreMesh(core_axis_name="c", subcore_axis_name="s")


@functools.partial(
    pl.kernel,
    out_type=(
        jax.ShapeDtypeStruct((B * 5,), jnp.float32),
        jax.ShapeDtypeStruct((B * 3,), jnp.float32),
        jax.ShapeDtypeStruct((B * 2,), jnp.float32),
    ),
    mesh=_mesh,
    scratch_types=(
        pltpu.VMEM((BPW,), jnp.int32),
        pltpu.VMEM((BPW,), jnp.int32),
        pltpu.VMEM((BPW,), jnp.float32),
        pltpu.VMEM((BPW,), jnp.float32),
        pltpu.VMEM((400,), jnp.float32),
        pltpu.VMEM((64,), jnp.float32),
        pltpu.VMEM((96,), jnp.float32),
        pltpu.VMEM((BPW * 5,), jnp.float32),
        pltpu.VMEM((BPW * 3,), jnp.float32),
        pltpu.VMEM((BPW * 2,), jnp.float32),
        pltpu.SemaphoreType.DMA,
    ),
    compiler_params=pltpu.CompilerParams(needs_layout_passes=False),
)
def _sc_embed(aid_h, iid_h, hp_h, atk_h, taba_h, tabi_h, wb_h,
              outa_h, outi_h, outs_h,
              aid_v, iid_v, hp_v, atk_v, taba_v, tabi_v, wb_v,
              outa_v, outi_v, outs_v, sem):
    wid = lax.axis_index("s") * NC + lax.axis_index("c")
    base = wid * BPW

    copies = [
        pltpu.async_copy(aid_h.at[pl.ds(base, BPW)], aid_v, sem),
        pltpu.async_copy(iid_h.at[pl.ds(base, BPW)], iid_v, sem),
        pltpu.async_copy(hp_h.at[pl.ds(base, BPW)], hp_v, sem),
        pltpu.async_copy(atk_h.at[pl.ds(base, BPW)], atk_v, sem),
        pltpu.async_copy(taba_h, taba_v, sem),
        pltpu.async_copy(tabi_h, tabi_v, sem),
        pltpu.async_copy(wb_h, wb_v, sem),
    ]
    for c in copies:
        c.wait()

    w00 = wb_v[pl.ds(0, L)]
    w01 = wb_v[pl.ds(L, L)]
    w10 = wb_v[pl.ds(2 * L, L)]
    w11 = wb_v[pl.ds(3 * L, L)]
    b0 = wb_v[pl.ds(4 * L, L)]
    b1 = wb_v[pl.ds(5 * L, L)]
    iota = lax.iota(jnp.int32, L)

    for g in range(GROUPS):
        off = g * L
        pos = iota + off
        aidx = aid_v[pl.ds(off, L)] * 5
        pa = pos * 5
        for j in range(5):
            plsc.store_scatter(outa_v, [pa + j],
                               plsc.load_gather(taba_v, [aidx + j]))
        iidx = iid_v[pl.ds(off, L)] * 3
        pi = pos * 3
        for j in range(3):
            plsc.store_scatter(outi_v, [pi + j],
                               plsc.load_gather(tabi_v, [iidx + j]))
        h = hp_v[pl.ds(off, L)]
        a = atk_v[pl.ds(off, L)]
        ps = pos * 2
        plsc.store_scatter(outs_v, [ps], h * w00 + a * w01 + b0)
        plsc.store_scatter(outs_v, [ps + 1], h * w10 + a * w11 + b1)

    out_copies = [
        pltpu.async_copy(outa_v, outa_h.at[pl.ds(base * 5, BPW * 5)], sem),
        pltpu.async_copy(outi_v, outi_h.at[pl.ds(base * 3, BPW * 3)], sem),
        pltpu.async_copy(outs_v, outs_h.at[pl.ds(base * 2, BPW * 2)], sem),
    ]
    for c in out_copies:
        c.wait()


def kernel(animal_id, item_id, hp, atk, emb_animal, emb_item, W_lin, b_lin):
    taba = emb_animal.reshape(-1)
    tabi = jnp.pad(emb_item.reshape(-1), (0, 4))
    wb = jnp.broadcast_to(
        jnp.concatenate([W_lin.reshape(-1), b_lin])[:, None], (6, L)
    ).reshape(-1)
    outa, outi, outs = _sc_embed(animal_id, item_id, hp, atk, taba, tabi, wb)
    return (outa.reshape(B, 5), outi.reshape(B, 3), outs.reshape(B, 2))


# speedup vs baseline: 2.0276x; 1.0172x over previous
import functools

import jax
import jax.numpy as jnp
from jax import lax
from jax.experimental import pallas as pl
from jax.experimental.pallas import tpu as pltpu
from jax.experimental.pallas import tpu_sc as plsc

B = 16384
NC, NS, L = 2, 16, 16
NW = NC * NS
BPW = B // NW
GROUPS = BPW // L

_mesh = plsc.VectorSubcoreMesh(core_axis_name="c", subcore_axis_name="s")


@functools.partial(
    pl.kernel,
    out_type=(
        jax.ShapeDtypeStruct((B * 5,), jnp.float32),
        jax.ShapeDtypeStruct((B * 3,), jnp.float32),
        jax.ShapeDtypeStruct((B * 2,), jnp.float32),
    ),
    mesh=_mesh,
    scratch_types=(
        pltpu.VMEM((BPW,), jnp.int32),
        pltpu.VMEM((BPW,), jnp.int32),
        pltpu.VMEM((BPW,), jnp.float32),
        pltpu.VMEM((BPW,), jnp.float32),
        pltpu.VMEM((400,), jnp.float32),
        pltpu.VMEM((64,), jnp.float32),
        pltpu.VMEM((96,), jnp.float32),
        pltpu.VMEM((BPW * 5,), jnp.float32),
        pltpu.VMEM((BPW * 3,), jnp.float32),
        pltpu.VMEM((BPW * 2,), jnp.float32),
        pltpu.SemaphoreType.DMA,
    ),
    compiler_params=pltpu.CompilerParams(needs_layout_passes=False),
)
def _sc_embed(aid_h, iid_h, hp_h, atk_h, taba_h, tabi_h, wb_h,
              outa_h, outi_h, outs_h,
              aid_v, iid_v, hp_v, atk_v, taba_v, tabi_v, wb_v,
              outa_v, outi_v, outs_v, sem):
    wid = lax.axis_index("s") * NC + lax.axis_index("c")
    base = wid * BPW

    copies = [
        pltpu.async_copy(aid_h.at[pl.ds(base, BPW)], aid_v, sem),
        pltpu.async_copy(iid_h.at[pl.ds(base, BPW)], iid_v, sem),
        pltpu.async_copy(hp_h.at[pl.ds(base, BPW)], hp_v, sem),
        pltpu.async_copy(atk_h.at[pl.ds(base, BPW)], atk_v, sem),
        pltpu.async_copy(taba_h, taba_v, sem),
        pltpu.async_copy(tabi_h, tabi_v, sem),
        pltpu.async_copy(wb_h, wb_v, sem),
    ]
    for c in copies:
        c.wait()

    w00 = wb_v[pl.ds(0, L)]
    w01 = wb_v[pl.ds(L, L)]
    w10 = wb_v[pl.ds(2 * L, L)]
    w11 = wb_v[pl.ds(3 * L, L)]
    b0 = wb_v[pl.ds(4 * L, L)]
    b1 = wb_v[pl.ds(5 * L, L)]
    iota = lax.iota(jnp.int32, L)

    pa0 = iota * 5
    pi0 = iota * 3
    ps0 = iota * 2
    for g in range(GROUPS):
        off = g * L
        aidx = aid_v[pl.ds(off, L)] * 5
        iidx = iid_v[pl.ds(off, L)] * 3
        h = hp_v[pl.ds(off, L)]
        a = atk_v[pl.ds(off, L)]
        ga = [plsc.load_gather(taba_v, [aidx + j]) for j in range(5)]
        gi = [plsc.load_gather(tabi_v, [iidx + j]) for j in range(3)]
        s0 = h * w00 + a * w01 + b0
        s1 = h * w10 + a * w11 + b1
        pa = pa0 + off * 5
        pi = pi0 + off * 3
        ps = ps0 + off * 2
        for j in range(5):
            plsc.store_scatter(outa_v, [pa + j], ga[j])
        for j in range(3):
            plsc.store_scatter(outi_v, [pi + j], gi[j])
        plsc.store_scatter(outs_v, [ps], s0)
        plsc.store_scatter(outs_v, [ps + 1], s1)

    out_copies = [
        pltpu.async_copy(outa_v, outa_h.at[pl.ds(base * 5, BPW * 5)], sem),
        pltpu.async_copy(outi_v, outi_h.at[pl.ds(base * 3, BPW * 3)], sem),
        pltpu.async_copy(outs_v, outs_h.at[pl.ds(base * 2, BPW * 2)], sem),
    ]
    for c in out_copies:
        c.wait()


def kernel(animal_id, item_id, hp, atk, emb_animal, emb_item, W_lin, b_lin):
    taba = emb_animal.reshape(-1)
    tabi = jnp.pad(emb_item.reshape(-1), (0, 4))
    wb = jnp.broadcast_to(
        jnp.concatenate([W_lin.reshape(-1), b_lin])[:, None], (6, L)
    ).reshape(-1)
    outa, outi, outs = _sc_embed(animal_id, item_id, hp, atk, taba, tabi, wb)
    return (outa.reshape(B, 5), outi.reshape(B, 3), outs.reshape(B, 2))


# speedup vs baseline: 2.0401x; 1.0061x over previous
import functools

import jax
import jax.numpy as jnp
from jax import lax
from jax.experimental import pallas as pl
from jax.experimental.pallas import tpu as pltpu
from jax.experimental.pallas import tpu_sc as plsc

B = 16384
NC, NS, L = 2, 16, 16
NW = NC * NS
BPW = B // NW
GROUPS = BPW // L

_mesh = plsc.VectorSubcoreMesh(core_axis_name="c", subcore_axis_name="s")


@functools.partial(
    pl.kernel,
    out_type=(
        jax.ShapeDtypeStruct((B * 5,), jnp.float32),
        jax.ShapeDtypeStruct((B * 3,), jnp.float32),
        jax.ShapeDtypeStruct((B * 2,), jnp.float32),
    ),
    mesh=_mesh,
    scratch_types=(
        pltpu.VMEM((BPW,), jnp.int32),
        pltpu.VMEM((BPW,), jnp.int32),
        pltpu.VMEM((BPW,), jnp.float32),
        pltpu.VMEM((BPW,), jnp.float32),
        pltpu.VMEM((400,), jnp.float32),
        pltpu.VMEM((64,), jnp.float32),
        pltpu.VMEM((96,), jnp.float32),
        pltpu.VMEM((BPW * 5,), jnp.float32),
        pltpu.VMEM((BPW * 3,), jnp.float32),
        pltpu.VMEM((BPW * 2,), jnp.float32),
        pltpu.SemaphoreType.DMA,
    ),
    compiler_params=pltpu.CompilerParams(needs_layout_passes=False),
)
def _sc_embed(aid_h, iid_h, hp_h, atk_h, taba_h, tabi_h, wb_h,
              outa_h, outi_h, outs_h,
              aid_v, iid_v, hp_v, atk_v, taba_v, tabi_v, wb_v,
              outa_v, outi_v, outs_v, sem):
    wid = lax.axis_index("s") * NC + lax.axis_index("c")
    base = wid * BPW

    copies = [
        pltpu.async_copy(aid_h.at[pl.ds(base, BPW)], aid_v, sem),
        pltpu.async_copy(iid_h.at[pl.ds(base, BPW)], iid_v, sem),
        pltpu.async_copy(hp_h.at[pl.ds(base, BPW)], hp_v, sem),
        pltpu.async_copy(atk_h.at[pl.ds(base, BPW)], atk_v, sem),
        pltpu.async_copy(taba_h, taba_v, sem),
        pltpu.async_copy(tabi_h, tabi_v, sem),
        pltpu.async_copy(wb_h, wb_v, sem),
    ]
    for c in copies:
        c.wait()

    w00 = wb_v[pl.ds(0, L)]
    w01 = wb_v[pl.ds(L, L)]
    w10 = wb_v[pl.ds(2 * L, L)]
    w11 = wb_v[pl.ds(3 * L, L)]
    b0 = wb_v[pl.ds(4 * L, L)]
    b1 = wb_v[pl.ds(5 * L, L)]
    iota = lax.iota(jnp.int32, L)

    pa0 = iota * 5
    pi0 = iota * 3
    ps0 = iota * 2
    CHUNK = 8
    out_copies = []
    for g in range(GROUPS):
        off = g * L
        aidx = aid_v[pl.ds(off, L)] * 5
        iidx = iid_v[pl.ds(off, L)] * 3
        h = hp_v[pl.ds(off, L)]
        a = atk_v[pl.ds(off, L)]
        ga = [plsc.load_gather(taba_v, [aidx + j]) for j in range(5)]
        gi = [plsc.load_gather(tabi_v, [iidx + j]) for j in range(3)]
        s0 = h * w00 + a * w01 + b0
        s1 = h * w10 + a * w11 + b1
        pa = pa0 + off * 5
        pi = pi0 + off * 3
        ps = ps0 + off * 2
        for j in range(5):
            plsc.store_scatter(outa_v, [pa + j], ga[j])
        for j in range(3):
            plsc.store_scatter(outi_v, [pi + j], gi[j])
        plsc.store_scatter(outs_v, [ps], s0)
        plsc.store_scatter(outs_v, [ps + 1], s1)
        if g % CHUNK == CHUNK - 1:
            lo = (g + 1 - CHUNK) * L
            n = CHUNK * L
            out_copies += [
                pltpu.async_copy(outa_v.at[pl.ds(lo * 5, n * 5)],
                                 outa_h.at[pl.ds(base * 5 + lo * 5, n * 5)],
                                 sem),
                pltpu.async_copy(outi_v.at[pl.ds(lo * 3, n * 3)],
                                 outi_h.at[pl.ds(base * 3 + lo * 3, n * 3)],
                                 sem),
                pltpu.async_copy(outs_v.at[pl.ds(lo * 2, n * 2)],
                                 outs_h.at[pl.ds(base * 2 + lo * 2, n * 2)],
                                 sem),
            ]

    for c in out_copies:
        c.wait()


def kernel(animal_id, item_id, hp, atk, emb_animal, emb_item, W_lin, b_lin):
    taba = emb_animal.reshape(-1)
    tabi = jnp.pad(emb_item.reshape(-1), (0, 4))
    wb = jnp.broadcast_to(
        jnp.concatenate([W_lin.reshape(-1), b_lin])[:, None], (6, L)
    ).reshape(-1)
    outa, outi, outs = _sc_embed(animal_id, item_id, hp, atk, taba, tabi, wb)
    return (outa.reshape(B, 5), outi.reshape(B, 3), outs.reshape(B, 2))
